# Initial kernel scaffold; baseline (speedup 1.0000x reference)
import jax
import jax.numpy as jnp
from jax.experimental import pallas as pl


def kernel(X, edge_index, W1, b1, W2, b2, W3, b3):
    raise NotImplementedError("write your pallas kernel here")


# speedup vs baseline: 1.7445x; 1.7445x over previous
import functools

import jax
import jax.numpy as jnp
from jax import lax
from jax.experimental import pallas as pl
from jax.experimental.pallas import tpu as pltpu
from jax.experimental.pallas import tpu_sc as plsc

N_NODES = 10000
N_EDGES = 320000
D = 128
N_RANGES = 16
N_HALVES = 2
N_REG = N_RANGES * N_HALVES
RPT = 640
N_PAD = N_RANGES * RPT
DUMMY_ROW = RPT
E_PAD = 323584
HALF_E = E_PAD // 2
CH2 = 2048
N_CH2 = E_PAD // CH2
SUB = 128
N_SUB = CH2 // SUB
EPT = E_PAD // 32
KB = 128
N_EB = EPT // KB
TOTAL_SLOTS = E_PAD + 64 * KB
PAD_DST = 1 << 30
NEG_INF = float("-inf")


_BLK = 1280


def _mm_first_body(x_ref, wa_ref, wb_ref, b_ref, a_ref, bo_ref):
    x = x_ref[...]
    a_ref[...] = (
        jnp.dot(x, wa_ref[...], preferred_element_type=jnp.float32,
                precision=lax.Precision.HIGHEST)
        + b_ref[...]
    )
    bo_ref[...] = jnp.dot(x, wb_ref[...], preferred_element_type=jnp.float32,
                          precision=lax.Precision.HIGHEST)


def _mm_mid_body(a_ref, s0_ref, s1_ref, wa_ref, wb_ref, b_ref,
                 a2_ref, b2_ref):
    s = jnp.maximum(s0_ref[...], s1_ref[...])
    h = jnp.where(s != -jnp.inf, jnp.maximum(a_ref[...] + s, 0.0), 0.0)
    a2_ref[...] = (
        jnp.dot(h, wa_ref[...], preferred_element_type=jnp.float32,
                precision=lax.Precision.HIGHEST)
        + b_ref[...]
    )
    b2_ref[...] = jnp.dot(h, wb_ref[...], preferred_element_type=jnp.float32,
                          precision=lax.Precision.HIGHEST)


def _mm_post_body(a_ref, s0_ref, s1_ref, w_ref, b_ref, o_ref):
    s = jnp.maximum(s0_ref[...], s1_ref[...])
    h = jnp.where(s != -jnp.inf, jnp.maximum(a_ref[...] + s, 0.0), 0.0)
    o_ref[...] = (
        jnp.dot(h, w_ref[...], preferred_element_type=jnp.float32,
                precision=lax.Precision.HIGHEST)
        + b_ref[...]
    )


def _row_spec():
    return pl.BlockSpec((_BLK, D), lambda i: (i, 0))


def _full_spec(shape):
    return pl.BlockSpec(shape, lambda i: (0,) * len(shape))


def _mm_first(x, wa, wb, b):
    return pl.pallas_call(
        _mm_first_body,
        grid=(N_PAD // _BLK,),
        in_specs=[_row_spec(), _full_spec((D, D)), _full_spec((D, D)),
                  _full_spec((1, D))],
        out_specs=[_row_spec(), _row_spec()],
        out_shape=[jax.ShapeDtypeStruct((N_PAD, D), jnp.float32)] * 2,
    )(x, wa, wb, b)


def _mm_mid(a, s2, wa, wb, b):
    return pl.pallas_call(
        _mm_mid_body,
        grid=(N_PAD // _BLK,),
        in_specs=[_row_spec(), _row_spec(), _row_spec(),
                  _full_spec((D, D)), _full_spec((D, D)), _full_spec((1, D))],
        out_specs=[_row_spec(), _row_spec()],
        out_shape=[jax.ShapeDtypeStruct((N_PAD, D), jnp.float32)] * 2,
    )(a, s2[:N_PAD], s2[N_PAD:], wa, wb, b)


def _mm_post(a, s2, w, b):
    return pl.pallas_call(
        _mm_post_body,
        grid=(N_PAD // _BLK,),
        in_specs=[_row_spec(), _row_spec(), _row_spec(),
                  _full_spec((D, D)), _full_spec((1, D))],
        out_specs=_row_spec(),
        out_shape=jax.ShapeDtypeStruct((N_PAD, D), jnp.float32),
    )(a, s2[:N_PAD], s2[N_PAD:], w, b)


def _pos_body(d_ref, pos_ref, base_ref, cnt_ref, carry):
    upper = jnp.where(
        lax.broadcasted_iota(jnp.int32, (SUB, SUB), 0)
        < lax.broadcasted_iota(jnp.int32, (SUB, SUB), 1),
        1.0, 0.0)
    tri64 = jnp.where(
        lax.broadcasted_iota(jnp.int32, (64, 64), 0)
        > lax.broadcasted_iota(jnp.int32, (64, 64), 1),
        1.0, 0.0)
    iota_k = lax.broadcasted_iota(jnp.int32, (64, SUB), 0)
    iota_e = lax.broadcasted_iota(jnp.int32, (SUB,), 0)

    def one_hot(d8, t, s):
        d = d8[s, :]
        epos = (8 * t + s) * SUB + iota_e
        h = (epos >= HALF_E).astype(jnp.int32)
        r = jnp.clip(d // RPT, 0, N_RANGES - 1)
        b = jnp.where(d >= N_PAD, 32, r * N_HALVES + h)
        return (b[None, :] == iota_k).astype(jnp.float32)

    n_grp = (N_CH2 * CH2) // (8 * SUB)
    carry[...] = jnp.zeros((64, 128), jnp.float32)

    def c1(t, _):
        d8 = d_ref[pl.ds(8 * t, 8), :]
        for s in range(8):
            oh = one_hot(d8, t, s)
            tot = jnp.sum(oh, axis=1, keepdims=True)
            carry[...] = carry[...] + jnp.broadcast_to(tot, (64, 128))
        return 0

    lax.fori_loop(0, n_grp, c1, 0)

    counts = carry[...]
    ci = counts.astype(jnp.int32)
    cr = (((ci + KB - 1) // KB) * KB).astype(jnp.float32)
    bases = jnp.dot(tri64, cr, precision=lax.Precision.HIGHEST)
    base_ref[...] = bases.astype(jnp.int32)
    cnt_ref[...] = ci

    carry[...] = bases

    def c2(t, _):
        d8 = d_ref[pl.ds(8 * t, 8), :]
        outs = []
        for s in range(8):
            oh = one_hot(d8, t, s)
            pre = jnp.dot(oh, upper,
                          precision=lax.Precision.HIGHEST)
            slot = jnp.sum(oh * (pre + carry[:, :SUB]), axis=0)
            outs.append(slot.astype(jnp.int32)[None, :])
            tot = jnp.sum(oh, axis=1, keepdims=True)
            carry[...] = carry[...] + jnp.broadcast_to(tot, (64, 128))
        pos_ref[pl.ds(8 * t, 8), :] = jnp.concatenate(outs, axis=0)
        return 0

    lax.fori_loop(0, n_grp, c2, 0)


def _edge_pos(d2):
    return pl.pallas_call(
        _pos_body,
        in_specs=[pl.BlockSpec((E_PAD // SUB, SUB), lambda: (0, 0))],
        out_specs=[pl.BlockSpec((E_PAD // SUB, SUB), lambda: (0, 0)),
                   pl.BlockSpec((64, 128), lambda: (0, 0)),
                   pl.BlockSpec((64, 128), lambda: (0, 0))],
        out_shape=[jax.ShapeDtypeStruct((E_PAD // SUB, SUB), jnp.int32),
                   jax.ShapeDtypeStruct((64, 128), jnp.int32),
                   jax.ShapeDtypeStruct((64, 128), jnp.int32)],
        scratch_shapes=[pltpu.VMEM((64, 128), jnp.float32)],
    )(d2)


def _scatter_kernel(src_hbm, dst_hbm, pos_hbm, b_hbm, rows_hbm, dlr_hbm,
                    srcv, dstv, posv, brow, dlbuf, sem):
    wid = lax.axis_index("c") * 16 + lax.axis_index("s")
    tbase = wid * EPT
    ones16 = jnp.full((16,), 1.0, dtype=jnp.float32)

    def batch(bi, _):
        off = tbase + bi * KB
        pltpu.sync_copy(src_hbm.at[pl.ds(off, KB)], srcv)
        pltpu.sync_copy(dst_hbm.at[pl.ds(off, KB)], dstv)
        pltpu.sync_copy(pos_hbm.at[pl.ds(off, KB)], posv)
        pltpu.async_copy(b_hbm.at[srcv], brow, sem).wait()

        def grp(j, _):
            dv = dstv[pl.ds(16 * j, 16)]
            dlv = (dv % RPT).astype(jnp.float32)
            for u in range(16):
                splat = ones16 * dlv[u]
                for f in range(D // 16):
                    dlbuf[16 * j + u, pl.ds(16 * f, 16)] = splat
            return 0

        lax.fori_loop(0, KB // 16, grp, 0)

        pltpu.async_copy(brow, rows_hbm.at[posv], sem).wait()
        pltpu.async_copy(dlbuf, dlr_hbm.at[posv], sem).wait()
        return 0

    lax.fori_loop(0, N_EB, batch, 0)


@functools.partial(
    pl.kernel,
    mesh=plsc.VectorSubcoreMesh(core_axis_name="c", subcore_axis_name="s"),
    out_type=[jax.ShapeDtypeStruct((TOTAL_SLOTS, D), jnp.float32),
              jax.ShapeDtypeStruct((TOTAL_SLOTS, D), jnp.float32)],
    scratch_types=[
        pltpu.VMEM((KB,), jnp.int32),
        pltpu.VMEM((KB,), jnp.int32),
        pltpu.VMEM((KB,), jnp.int32),
        pltpu.VMEM((KB, D), jnp.float32),
        pltpu.VMEM((KB, D), jnp.float32),
        pltpu.SemaphoreType.DMA,
    ],
)
def _edge_scatter(src_hbm, dst_hbm, pos_hbm, b_hbm, rows_hbm, dlr_hbm,
                  *scratch):
    _scatter_kernel(src_hbm, dst_hbm, pos_hbm, b_hbm, rows_hbm, dlr_hbm,
                    *scratch)


def _apply_kernel(rows_hbm, dlr_hbm, base_hbm, cnt_hbm, out_hbm,
                  acc, rbuf, dbuf, meta):
    wid = lax.axis_index("c") * 16 + lax.axis_index("s")
    rng = wid % N_RANGES
    half = wid // N_RANGES
    region = rng * N_HALVES + half

    pltpu.sync_copy(base_hbm.at[pl.ds(region, 1), :], meta.at[pl.ds(0, 1), :])
    pltpu.sync_copy(cnt_hbm.at[pl.ds(region, 1), :], meta.at[pl.ds(1, 1), :])
    base = pl.multiple_of(meta[0, pl.ds(0, 16)][0], KB)
    count = meta[1, pl.ds(0, 16)][0]

    neg = jnp.full((16,), NEG_INF, dtype=jnp.float32)

    def init_body(r, _):
        for f in range(D // 16):
            acc[r, pl.ds(16 * f, 16)] = neg
        return 0

    lax.fori_loop(0, RPT + 8, init_body, 0)

    def batch(bi, _):
        roff = base + bi * KB
        pltpu.sync_copy(rows_hbm.at[pl.ds(roff, KB), :], rbuf)
        pltpu.sync_copy(dlr_hbm.at[pl.ds(roff, KB), :], dbuf)
        sbase = bi * KB

        def grp(j, _):
            for u in range(16):
                e = 16 * j + u
                valid = sbase + e < count
                dlf = dbuf[e, pl.ds(0, 16)][0]
                dl = jnp.clip(dlf.astype(jnp.int32), 0, DUMMY_ROW)
                dl = jnp.where(valid, dl, DUMMY_ROW)
                for f in range(D // 16):
                    sl = pl.ds(16 * f, 16)
                    v = jnp.where(valid, rbuf[e, sl], neg)
                    acc[dl, sl] = jnp.maximum(acc[dl, sl], v)
            return 0

        lax.fori_loop(0, KB // 16, grp, 0)
        return 0

    n_b = (count + KB - 1) // KB
    lax.fori_loop(0, n_b, batch, 0)

    pltpu.sync_copy(acc.at[pl.ds(0, RPT)],
                    out_hbm.at[pl.ds(half * N_PAD + rng * RPT, RPT)])


@functools.partial(
    pl.kernel,
    mesh=plsc.VectorSubcoreMesh(core_axis_name="c", subcore_axis_name="s"),
    out_type=jax.ShapeDtypeStruct((N_HALVES * N_PAD, D), jnp.float32),
    scratch_types=[
        pltpu.VMEM((RPT + 8, D), jnp.float32),
        pltpu.VMEM((KB, D), jnp.float32),
        pltpu.VMEM((KB, D), jnp.float32),
        pltpu.VMEM((2, D), jnp.int32),
        pltpu.SemaphoreType.DMA,
    ],
)
def _seg_apply(rows_hbm, dlr_hbm, base_hbm, cnt_hbm, out_hbm,
               acc, rbuf, dbuf, meta, sem):
    del sem
    _apply_kernel(rows_hbm, dlr_hbm, base_hbm, cnt_hbm, out_hbm,
                  acc, rbuf, dbuf, meta)


def kernel(X, edge_index, W1, b1, W2, b2, W3, b3):
    src = edge_index[0]
    dst = edge_index[1]

    srcp = jnp.zeros((E_PAD,), jnp.int32).at[:N_EDGES].set(src)
    dstp = jnp.full((E_PAD,), PAD_DST, jnp.int32).at[:N_EDGES].set(dst)

    w1a = W1[:D] - W1[D:]
    w1b = W1[D:]
    w2a = W2[:D] - W2[D:]
    w2b = W2[D:]
    w3p = jnp.pad(W3, ((0, 0), (0, D - W3.shape[1])))
    b3p = jnp.pad(b3, (0, D - b3.shape[0]))

    xp = jnp.pad(X, ((0, N_PAD - N_NODES), (0, 0)))

    pos2, bases, cnts = _edge_pos(dstp.reshape(E_PAD // SUB, SUB))
    pos = pos2.reshape(-1)

    a1, bb1 = _mm_first(xp, w1a, w1b, b1.reshape(1, D))
    rows1, dlr1 = _edge_scatter(srcp, dstp, pos, bb1)
    s1 = _seg_apply(rows1, dlr1, bases, cnts)
    a2, bb2 = _mm_mid(a1, s1, w2a, w2b, b2.reshape(1, D))
    rows2, dlr2 = _edge_scatter(srcp, dstp, pos, bb2)
    s2 = _seg_apply(rows2, dlr2, bases, cnts)
    out = _mm_post(a2, s2, w3p, b3p.reshape(1, D))
    return out[:N_NODES, :W3.shape[1]]


# speedup vs baseline: 1.9210x; 1.1012x over previous
import functools

import jax
import jax.numpy as jnp
from jax import lax
from jax.experimental import pallas as pl
from jax.experimental.pallas import tpu as pltpu
from jax.experimental.pallas import tpu_sc as plsc

N_NODES = 10000
N_EDGES = 320000
D = 128
N_RANGES = 16
N_HALVES = 2
N_REG = N_RANGES * N_HALVES
RPT = 640
N_PAD = N_RANGES * RPT
DUMMY_ROW = RPT
E_PAD = 323584
HALF_E = E_PAD // 2
CH2 = 2048
N_CH2 = E_PAD // CH2
SUB = 128
N_SUB = CH2 // SUB
EPT = E_PAD // 32
KB = 128
N_EB = EPT // KB
TOTAL_SLOTS = E_PAD + 64 * KB
PAD_DST = 1 << 30
NEG_INF = float("-inf")


_BLK = 1280


def _mm_first_body(x_ref, wa_ref, wb_ref, b_ref, a_ref, bo_ref):
    x = x_ref[...]
    a_ref[...] = (
        jnp.dot(x, wa_ref[...], preferred_element_type=jnp.float32,
                precision=lax.Precision.HIGHEST)
        + b_ref[...]
    )
    bo_ref[...] = jnp.dot(x, wb_ref[...], preferred_element_type=jnp.float32,
                          precision=lax.Precision.HIGHEST)


def _mm_mid_body(a_ref, s0_ref, s1_ref, wa_ref, wb_ref, b_ref,
                 a2_ref, b2_ref):
    s = jnp.maximum(s0_ref[...], s1_ref[...])
    h = jnp.where(s != -jnp.inf, jnp.maximum(a_ref[...] + s, 0.0), 0.0)
    a2_ref[...] = (
        jnp.dot(h, wa_ref[...], preferred_element_type=jnp.float32,
                precision=lax.Precision.HIGHEST)
        + b_ref[...]
    )
    b2_ref[...] = jnp.dot(h, wb_ref[...], preferred_element_type=jnp.float32,
                          precision=lax.Precision.HIGHEST)


def _mm_post_body(a_ref, s0_ref, s1_ref, w_ref, b_ref, o_ref):
    s = jnp.maximum(s0_ref[...], s1_ref[...])
    h = jnp.where(s != -jnp.inf, jnp.maximum(a_ref[...] + s, 0.0), 0.0)
    o_ref[...] = (
        jnp.dot(h, w_ref[...], preferred_element_type=jnp.float32,
                precision=lax.Precision.HIGHEST)
        + b_ref[...]
    )


def _row_spec():
    return pl.BlockSpec((_BLK, D), lambda i: (i, 0))


def _full_spec(shape):
    return pl.BlockSpec(shape, lambda i: (0,) * len(shape))


def _mm_first(x, wa, wb, b):
    return pl.pallas_call(
        _mm_first_body,
        grid=(N_PAD // _BLK,),
        in_specs=[_row_spec(), _full_spec((D, D)), _full_spec((D, D)),
                  _full_spec((1, D))],
        out_specs=[_row_spec(), _row_spec()],
        out_shape=[jax.ShapeDtypeStruct((N_PAD, D), jnp.float32)] * 2,
    )(x, wa, wb, b)


def _mm_mid(a, s2, wa, wb, b):
    return pl.pallas_call(
        _mm_mid_body,
        grid=(N_PAD // _BLK,),
        in_specs=[_row_spec(), _row_spec(), _row_spec(),
                  _full_spec((D, D)), _full_spec((D, D)), _full_spec((1, D))],
        out_specs=[_row_spec(), _row_spec()],
        out_shape=[jax.ShapeDtypeStruct((N_PAD, D), jnp.float32)] * 2,
    )(a, s2[:N_PAD], s2[N_PAD:], wa, wb, b)


def _mm_post(a, s2, w, b):
    return pl.pallas_call(
        _mm_post_body,
        grid=(N_PAD // _BLK,),
        in_specs=[_row_spec(), _row_spec(), _row_spec(),
                  _full_spec((D, D)), _full_spec((1, D))],
        out_specs=_row_spec(),
        out_shape=jax.ShapeDtypeStruct((N_PAD, D), jnp.float32),
    )(a, s2[:N_PAD], s2[N_PAD:], w, b)


def _pos_body(d_ref, pos_ref, base_ref, cnt_ref, carry):
    upper = jnp.where(
        lax.broadcasted_iota(jnp.int32, (SUB, SUB), 0)
        < lax.broadcasted_iota(jnp.int32, (SUB, SUB), 1),
        1.0, 0.0)
    tri64 = jnp.where(
        lax.broadcasted_iota(jnp.int32, (64, 64), 0)
        > lax.broadcasted_iota(jnp.int32, (64, 64), 1),
        1.0, 0.0)
    iota_k = lax.broadcasted_iota(jnp.int32, (64, SUB), 0)
    iota_e = lax.broadcasted_iota(jnp.int32, (SUB,), 0)

    def one_hot(d8, t, s):
        d = d8[s, :]
        epos = (8 * t + s) * SUB + iota_e
        h = (epos >= HALF_E).astype(jnp.int32)
        r = jnp.clip(d // RPT, 0, N_RANGES - 1)
        b = jnp.where(d >= N_PAD, 32, r * N_HALVES + h)
        return (b[None, :] == iota_k).astype(jnp.float32)

    n_grp = (N_CH2 * CH2) // (8 * SUB)
    carry[...] = jnp.zeros((64, 128), jnp.float32)

    def c1(t, _):
        d8 = d_ref[pl.ds(8 * t, 8), :]
        for s in range(8):
            oh = one_hot(d8, t, s)
            tot = jnp.sum(oh, axis=1, keepdims=True)
            carry[...] = carry[...] + jnp.broadcast_to(tot, (64, 128))
        return 0

    lax.fori_loop(0, n_grp, c1, 0)

    counts = carry[...]
    ci = counts.astype(jnp.int32)
    cr = (((ci + KB - 1) // KB) * KB).astype(jnp.float32)
    bases = jnp.dot(tri64, cr)
    base_ref[...] = bases.astype(jnp.int32)
    cnt_ref[...] = ci

    carry[...] = bases

    def c2(t, _):
        d8 = d_ref[pl.ds(8 * t, 8), :]
        outs = []
        for s in range(8):
            oh = one_hot(d8, t, s)
            pre = jnp.dot(oh, upper)
            slot = jnp.sum(oh * (pre + carry[:, :SUB]), axis=0)
            outs.append(slot.astype(jnp.int32)[None, :])
            tot = jnp.sum(oh, axis=1, keepdims=True)
            carry[...] = carry[...] + jnp.broadcast_to(tot, (64, 128))
        pos_ref[pl.ds(8 * t, 8), :] = jnp.concatenate(outs, axis=0)
        return 0

    lax.fori_loop(0, n_grp, c2, 0)


def _edge_pos(d2):
    return pl.pallas_call(
        _pos_body,
        in_specs=[pl.BlockSpec((E_PAD // SUB, SUB), lambda: (0, 0))],
        out_specs=[pl.BlockSpec((E_PAD // SUB, SUB), lambda: (0, 0)),
                   pl.BlockSpec((64, 128), lambda: (0, 0)),
                   pl.BlockSpec((64, 128), lambda: (0, 0))],
        out_shape=[jax.ShapeDtypeStruct((E_PAD // SUB, SUB), jnp.int32),
                   jax.ShapeDtypeStruct((64, 128), jnp.int32),
                   jax.ShapeDtypeStruct((64, 128), jnp.int32)],
        scratch_shapes=[pltpu.VMEM((64, 128), jnp.float32)],
    )(d2)


def _scatter_kernel(src_hbm, dst_hbm, pos_hbm, b_hbm, rows_hbm, dlr_hbm,
                    srcv, dstv, posv, brow, dlbuf, sem):
    wid = lax.axis_index("c") * 16 + lax.axis_index("s")
    tbase = wid * EPT
    ones16 = jnp.full((16,), 1.0, dtype=jnp.float32)

    def batch(bi, _):
        off = tbase + bi * KB
        pltpu.sync_copy(src_hbm.at[pl.ds(off, KB)], srcv)
        pltpu.sync_copy(dst_hbm.at[pl.ds(off, KB)], dstv)
        pltpu.sync_copy(pos_hbm.at[pl.ds(off, KB)], posv)
        pltpu.async_copy(b_hbm.at[srcv], brow, sem).wait()

        def grp(j, _):
            dv = dstv[pl.ds(16 * j, 16)]
            dlv = (dv % RPT).astype(jnp.float32)
            for u in range(16):
                splat = ones16 * dlv[u]
                for f in range(D // 16):
                    dlbuf[16 * j + u, pl.ds(16 * f, 16)] = splat
            return 0

        lax.fori_loop(0, KB // 16, grp, 0)

        c1 = pltpu.async_copy(brow, rows_hbm.at[posv], sem)
        c2 = pltpu.async_copy(dlbuf, dlr_hbm.at[posv], sem)
        c1.wait()
        c2.wait()
        return 0

    lax.fori_loop(0, N_EB, batch, 0)


@functools.partial(
    pl.kernel,
    mesh=plsc.VectorSubcoreMesh(core_axis_name="c", subcore_axis_name="s"),
    out_type=[jax.ShapeDtypeStruct((TOTAL_SLOTS, D), jnp.float32),
              jax.ShapeDtypeStruct((TOTAL_SLOTS, D), jnp.float32)],
    scratch_types=[
        pltpu.VMEM((KB,), jnp.int32),
        pltpu.VMEM((KB,), jnp.int32),
        pltpu.VMEM((KB,), jnp.int32),
        pltpu.VMEM((KB, D), jnp.float32),
        pltpu.VMEM((KB, D), jnp.float32),
        pltpu.SemaphoreType.DMA,
    ],
)
def _edge_scatter(src_hbm, dst_hbm, pos_hbm, b_hbm, rows_hbm, dlr_hbm,
                  *scratch):
    _scatter_kernel(src_hbm, dst_hbm, pos_hbm, b_hbm, rows_hbm, dlr_hbm,
                    *scratch)


def _scatter_rows_kernel(src_hbm, pos_hbm, b_hbm, rows_hbm,
                         srcv, posv, brow, sem):
    wid = lax.axis_index("c") * 16 + lax.axis_index("s")
    tbase = wid * EPT

    def batch(bi, _):
        off = tbase + bi * KB
        pltpu.sync_copy(src_hbm.at[pl.ds(off, KB)], srcv)
        pltpu.sync_copy(pos_hbm.at[pl.ds(off, KB)], posv)
        pltpu.async_copy(b_hbm.at[srcv], brow, sem).wait()
        pltpu.async_copy(brow, rows_hbm.at[posv], sem).wait()
        return 0

    lax.fori_loop(0, N_EB, batch, 0)


@functools.partial(
    pl.kernel,
    mesh=plsc.VectorSubcoreMesh(core_axis_name="c", subcore_axis_name="s"),
    out_type=jax.ShapeDtypeStruct((TOTAL_SLOTS, D), jnp.float32),
    scratch_types=[
        pltpu.VMEM((KB,), jnp.int32),
        pltpu.VMEM((KB,), jnp.int32),
        pltpu.VMEM((KB, D), jnp.float32),
        pltpu.SemaphoreType.DMA,
    ],
)
def _edge_scatter_rows(src_hbm, pos_hbm, b_hbm, rows_hbm, *scratch):
    _scatter_rows_kernel(src_hbm, pos_hbm, b_hbm, rows_hbm, *scratch)


def _apply_kernel(rows_hbm, dlr_hbm, base_hbm, cnt_hbm, out_hbm,
                  acc, rbuf, dbuf, meta):
    wid = lax.axis_index("c") * 16 + lax.axis_index("s")
    rng = wid % N_RANGES
    half = wid // N_RANGES
    region = rng * N_HALVES + half

    pltpu.sync_copy(base_hbm.at[pl.ds(region, 1), :], meta.at[pl.ds(0, 1), :])
    pltpu.sync_copy(cnt_hbm.at[pl.ds(region, 1), :], meta.at[pl.ds(1, 1), :])
    base = pl.multiple_of(meta[0, pl.ds(0, 16)][0], KB)
    count = meta[1, pl.ds(0, 16)][0]

    neg = jnp.full((16,), NEG_INF, dtype=jnp.float32)

    def init_body(r, _):
        for f in range(D // 16):
            acc[r, pl.ds(16 * f, 16)] = neg
        return 0

    lax.fori_loop(0, RPT + 8, init_body, 0)

    def batch(bi, _):
        roff = base + bi * KB
        pltpu.sync_copy(rows_hbm.at[pl.ds(roff, KB), :], rbuf)
        pltpu.sync_copy(dlr_hbm.at[pl.ds(roff, KB), :], dbuf)
        sbase = bi * KB

        def grp(j, _):
            for u in range(16):
                e = 16 * j + u
                valid = sbase + e < count
                dlf = dbuf[e, pl.ds(0, 16)][0]
                dl = jnp.clip(dlf.astype(jnp.int32), 0, DUMMY_ROW)
                dl = jnp.where(valid, dl, DUMMY_ROW)
                for f in range(D // 16):
                    sl = pl.ds(16 * f, 16)
                    v = jnp.where(valid, rbuf[e, sl], neg)
                    acc[dl, sl] = jnp.maximum(acc[dl, sl], v)
            return 0

        lax.fori_loop(0, KB // 16, grp, 0)
        return 0

    n_b = (count + KB - 1) // KB
    lax.fori_loop(0, n_b, batch, 0)

    pltpu.sync_copy(acc.at[pl.ds(0, RPT)],
                    out_hbm.at[pl.ds(half * N_PAD + rng * RPT, RPT)])


@functools.partial(
    pl.kernel,
    mesh=plsc.VectorSubcoreMesh(core_axis_name="c", subcore_axis_name="s"),
    out_type=jax.ShapeDtypeStruct((N_HALVES * N_PAD, D), jnp.float32),
    scratch_types=[
        pltpu.VMEM((RPT + 8, D), jnp.float32),
        pltpu.VMEM((KB, D), jnp.float32),
        pltpu.VMEM((KB, D), jnp.float32),
        pltpu.VMEM((2, D), jnp.int32),
        pltpu.SemaphoreType.DMA,
    ],
)
def _seg_apply(rows_hbm, dlr_hbm, base_hbm, cnt_hbm, out_hbm,
               acc, rbuf, dbuf, meta, sem):
    del sem
    _apply_kernel(rows_hbm, dlr_hbm, base_hbm, cnt_hbm, out_hbm,
                  acc, rbuf, dbuf, meta)


def kernel(X, edge_index, W1, b1, W2, b2, W3, b3):
    src = edge_index[0]
    dst = edge_index[1]

    srcp = jnp.zeros((E_PAD,), jnp.int32).at[:N_EDGES].set(src)
    dstp = jnp.full((E_PAD,), PAD_DST, jnp.int32).at[:N_EDGES].set(dst)

    w1a = W1[:D] - W1[D:]
    w1b = W1[D:]
    w2a = W2[:D] - W2[D:]
    w2b = W2[D:]
    w3p = jnp.pad(W3, ((0, 0), (0, D - W3.shape[1])))
    b3p = jnp.pad(b3, (0, D - b3.shape[0]))

    xp = jnp.pad(X, ((0, N_PAD - N_NODES), (0, 0)))

    pos2, bases, cnts = _edge_pos(dstp.reshape(E_PAD // SUB, SUB))
    pos = pos2.reshape(-1)

    a1, bb1 = _mm_first(xp, w1a, w1b, b1.reshape(1, D))
    rows1, dlr1 = _edge_scatter(srcp, dstp, pos, bb1)
    s1 = _seg_apply(rows1, dlr1, bases, cnts)
    a2, bb2 = _mm_mid(a1, s1, w2a, w2b, b2.reshape(1, D))
    rows2 = _edge_scatter_rows(srcp, pos, bb2)
    s2 = _seg_apply(rows2, dlr1, bases, cnts)
    out = _mm_post(a2, s2, w3p, b3p.reshape(1, D))
    return out[:N_NODES, :W3.shape[1]]


# speedup vs baseline: 2.3928x; 1.2456x over previous
import functools

import jax
import jax.numpy as jnp
from jax import lax
from jax.experimental import pallas as pl
from jax.experimental.pallas import tpu as pltpu
from jax.experimental.pallas import tpu_sc as plsc

N_NODES = 10000
N_EDGES = 320000
D = 128
N_RANGES = 16
N_HALVES = 2
N_REG = N_RANGES * N_HALVES
RPT = 640
N_PAD = N_RANGES * RPT
DUMMY_ROW = RPT
E_PAD = 323584
HALF_E = E_PAD // 2
CH2 = 2048
N_CH2 = E_PAD // CH2
SUB = 128
N_SUB = CH2 // SUB
EPT = E_PAD // 32
KB = 128
N_EB = EPT // KB
TOTAL_SLOTS = E_PAD + 64 * KB
PAD_DST = 1 << 30
NEG_INF = float("-inf")


_BLK = 1280


def _mm_first_body(x_ref, wa_ref, wb_ref, b_ref, a_ref, bo_ref):
    x = x_ref[...]
    a_ref[...] = (
        jnp.dot(x, wa_ref[...], preferred_element_type=jnp.float32,
                precision=lax.Precision.HIGHEST)
        + b_ref[...]
    )
    bo_ref[...] = jnp.dot(x, wb_ref[...], preferred_element_type=jnp.float32,
                          precision=lax.Precision.HIGHEST)


def _mm_mid_body(a_ref, s0_ref, s1_ref, wa_ref, wb_ref, b_ref,
                 a2_ref, b2_ref):
    s = jnp.maximum(s0_ref[...], s1_ref[...])
    h = jnp.where(s != -jnp.inf, jnp.maximum(a_ref[...] + s, 0.0), 0.0)
    a2_ref[...] = (
        jnp.dot(h, wa_ref[...], preferred_element_type=jnp.float32,
                precision=lax.Precision.HIGHEST)
        + b_ref[...]
    )
    b2_ref[...] = jnp.dot(h, wb_ref[...], preferred_element_type=jnp.float32,
                          precision=lax.Precision.HIGHEST)


def _mm_post_body(a_ref, s0_ref, s1_ref, w_ref, b_ref, o_ref):
    s = jnp.maximum(s0_ref[...], s1_ref[...])
    h = jnp.where(s != -jnp.inf, jnp.maximum(a_ref[...] + s, 0.0), 0.0)
    o_ref[...] = (
        jnp.dot(h, w_ref[...], preferred_element_type=jnp.float32,
                precision=lax.Precision.HIGHEST)
        + b_ref[...]
    )


def _row_spec():
    return pl.BlockSpec((_BLK, D), lambda i: (i, 0))


def _full_spec(shape):
    return pl.BlockSpec(shape, lambda i: (0,) * len(shape))


def _mm_first(x, wa, wb, b):
    return pl.pallas_call(
        _mm_first_body,
        grid=(N_PAD // _BLK,),
        in_specs=[_row_spec(), _full_spec((D, D)), _full_spec((D, D)),
                  _full_spec((1, D))],
        out_specs=[_row_spec(), _row_spec()],
        out_shape=[jax.ShapeDtypeStruct((N_PAD, D), jnp.float32)] * 2,
    )(x, wa, wb, b)


def _mm_mid(a, s2, wa, wb, b):
    return pl.pallas_call(
        _mm_mid_body,
        grid=(N_PAD // _BLK,),
        in_specs=[_row_spec(), _row_spec(), _row_spec(),
                  _full_spec((D, D)), _full_spec((D, D)), _full_spec((1, D))],
        out_specs=[_row_spec(), _row_spec()],
        out_shape=[jax.ShapeDtypeStruct((N_PAD, D), jnp.float32)] * 2,
    )(a, s2[:N_PAD], s2[N_PAD:], wa, wb, b)


def _mm_post(a, s2, w, b):
    return pl.pallas_call(
        _mm_post_body,
        grid=(N_PAD // _BLK,),
        in_specs=[_row_spec(), _row_spec(), _row_spec(),
                  _full_spec((D, D)), _full_spec((1, D))],
        out_specs=_row_spec(),
        out_shape=jax.ShapeDtypeStruct((N_PAD, D), jnp.float32),
    )(a, s2[:N_PAD], s2[N_PAD:], w, b)


def _pos_body(d_ref, pos_ref, base_ref, cnt_ref, carry):
    upper = jnp.where(
        lax.broadcasted_iota(jnp.int32, (SUB, SUB), 0)
        < lax.broadcasted_iota(jnp.int32, (SUB, SUB), 1),
        1.0, 0.0)
    tri64 = jnp.where(
        lax.broadcasted_iota(jnp.int32, (64, 64), 0)
        > lax.broadcasted_iota(jnp.int32, (64, 64), 1),
        1.0, 0.0)
    iota_k = lax.broadcasted_iota(jnp.int32, (64, SUB), 0)
    iota_e = lax.broadcasted_iota(jnp.int32, (SUB,), 0)

    def one_hot(d8, t, s):
        d = d8[s, :]
        epos = (8 * t + s) * SUB + iota_e
        h = (epos >= HALF_E).astype(jnp.int32)
        r = jnp.clip(d // RPT, 0, N_RANGES - 1)
        b = jnp.where(d >= N_PAD, 32, r * N_HALVES + h)
        return (b[None, :] == iota_k).astype(jnp.float32)

    n_grp = (N_CH2 * CH2) // (8 * SUB)
    carry[...] = jnp.zeros((64, 128), jnp.float32)

    def c1(t, _):
        d8 = d_ref[pl.ds(8 * t, 8), :]
        for s in range(8):
            oh = one_hot(d8, t, s)
            tot = jnp.sum(oh, axis=1, keepdims=True)
            carry[...] = carry[...] + jnp.broadcast_to(tot, (64, 128))
        return 0

    lax.fori_loop(0, n_grp, c1, 0)

    counts = carry[...]
    ci = counts.astype(jnp.int32)
    cr = (((ci + KB - 1) // KB) * KB).astype(jnp.float32)
    bases = jnp.dot(tri64, cr)
    base_ref[...] = bases.astype(jnp.int32)
    cnt_ref[...] = ci

    carry[...] = bases

    def c2(t, _):
        d8 = d_ref[pl.ds(8 * t, 8), :]
        outs = []
        for s in range(8):
            oh = one_hot(d8, t, s)
            pre = jnp.dot(oh, upper)
            slot = jnp.sum(oh * (pre + carry[:, :SUB]), axis=0)
            outs.append(slot.astype(jnp.int32)[None, :])
            tot = jnp.sum(oh, axis=1, keepdims=True)
            carry[...] = carry[...] + jnp.broadcast_to(tot, (64, 128))
        pos_ref[pl.ds(8 * t, 8), :] = jnp.concatenate(outs, axis=0)
        return 0

    lax.fori_loop(0, n_grp, c2, 0)


def _edge_pos(d2):
    return pl.pallas_call(
        _pos_body,
        in_specs=[pl.BlockSpec((E_PAD // SUB, SUB), lambda: (0, 0))],
        out_specs=[pl.BlockSpec((E_PAD // SUB, SUB), lambda: (0, 0)),
                   pl.BlockSpec((64, 128), lambda: (0, 0)),
                   pl.BlockSpec((64, 128), lambda: (0, 0))],
        out_shape=[jax.ShapeDtypeStruct((E_PAD // SUB, SUB), jnp.int32),
                   jax.ShapeDtypeStruct((64, 128), jnp.int32),
                   jax.ShapeDtypeStruct((64, 128), jnp.int32)],
        scratch_shapes=[pltpu.VMEM((64, 128), jnp.float32)],
    )(d2)


def _scatter_kernel(src_hbm, dst_hbm, pos_hbm, b_hbm, rows_hbm, dlr_hbm,
                    srcv, dstv, posv, brow, dlbuf, sem):
    wid = lax.axis_index("c") * 16 + lax.axis_index("s")
    tbase = wid * EPT
    ones16 = jnp.full((16,), 1.0, dtype=jnp.float32)

    def batch(bi, _):
        off = tbase + bi * KB
        pltpu.sync_copy(src_hbm.at[pl.ds(off, KB)], srcv)
        pltpu.sync_copy(dst_hbm.at[pl.ds(off, KB)], dstv)
        pltpu.sync_copy(pos_hbm.at[pl.ds(off, KB)], posv)
        g = pltpu.async_copy(b_hbm.at[srcv], brow, sem)

        def grp(j, _):
            dv = dstv[pl.ds(16 * j, 16)]
            dlv = (dv % RPT).astype(jnp.float32)
            for u in range(16):
                splat = ones16 * dlv[u]
                for f in range(D // 16):
                    dlbuf[16 * j + u, pl.ds(16 * f, 16)] = splat
            return 0

        lax.fori_loop(0, KB // 16, grp, 0)
        g.wait()

        c1 = pltpu.async_copy(brow, rows_hbm.at[posv], sem)
        c2 = pltpu.async_copy(dlbuf, dlr_hbm.at[posv], sem)
        c1.wait()
        c2.wait()
        return 0

    lax.fori_loop(0, N_EB, batch, 0)


@functools.partial(
    pl.kernel,
    mesh=plsc.VectorSubcoreMesh(core_axis_name="c", subcore_axis_name="s"),
    out_type=[jax.ShapeDtypeStruct((TOTAL_SLOTS, D), jnp.float32),
              jax.ShapeDtypeStruct((TOTAL_SLOTS, D), jnp.float32)],
    scratch_types=[
        pltpu.VMEM((KB,), jnp.int32),
        pltpu.VMEM((KB,), jnp.int32),
        pltpu.VMEM((KB,), jnp.int32),
        pltpu.VMEM((KB, D), jnp.float32),
        pltpu.VMEM((KB, D), jnp.float32),
        pltpu.SemaphoreType.DMA,
    ],
)
def _edge_scatter(src_hbm, dst_hbm, pos_hbm, b_hbm, rows_hbm, dlr_hbm,
                  *scratch):
    _scatter_kernel(src_hbm, dst_hbm, pos_hbm, b_hbm, rows_hbm, dlr_hbm,
                    *scratch)


def _scatter_rows_kernel(src_hbm, pos_hbm, b_hbm, rows_hbm,
                         srcv, posv, brow, sem):
    wid = lax.axis_index("c") * 16 + lax.axis_index("s")
    tbase = wid * EPT

    def batch(bi, _):
        off = tbase + bi * KB
        pltpu.sync_copy(src_hbm.at[pl.ds(off, KB)], srcv)
        pltpu.sync_copy(pos_hbm.at[pl.ds(off, KB)], posv)
        pltpu.async_copy(b_hbm.at[srcv], brow, sem).wait()
        pltpu.async_copy(brow, rows_hbm.at[posv], sem).wait()
        return 0

    lax.fori_loop(0, N_EB, batch, 0)


@functools.partial(
    pl.kernel,
    mesh=plsc.VectorSubcoreMesh(core_axis_name="c", subcore_axis_name="s"),
    out_type=jax.ShapeDtypeStruct((TOTAL_SLOTS, D), jnp.float32),
    scratch_types=[
        pltpu.VMEM((KB,), jnp.int32),
        pltpu.VMEM((KB,), jnp.int32),
        pltpu.VMEM((KB, D), jnp.float32),
        pltpu.SemaphoreType.DMA,
    ],
)
def _edge_scatter_rows(src_hbm, pos_hbm, b_hbm, rows_hbm, *scratch):
    _scatter_rows_kernel(src_hbm, pos_hbm, b_hbm, rows_hbm, *scratch)


KA = 64


def _apply_kernel(rows_hbm, dlr_hbm, base_hbm, cnt_hbm, out_hbm,
                  acc, rb0, db0, rb1, db1, meta, sem):
    wid = lax.axis_index("c") * 16 + lax.axis_index("s")
    rng = wid % N_RANGES
    half = wid // N_RANGES
    region = rng * N_HALVES + half

    pltpu.sync_copy(base_hbm.at[pl.ds(region, 1), :], meta.at[pl.ds(0, 1), :])
    pltpu.sync_copy(cnt_hbm.at[pl.ds(region, 1), :], meta.at[pl.ds(1, 1), :])
    base = pl.multiple_of(meta[0, pl.ds(0, 16)][0], KB)
    count = meta[1, pl.ds(0, 16)][0]

    neg = jnp.full((16,), NEG_INF, dtype=jnp.float32)

    def init_body(r, _):
        for f in range(D // 16):
            acc[r, pl.ds(16 * f, 16)] = neg
        return 0

    lax.fori_loop(0, RPT + 8, init_body, 0)

    n_b = (count + KA - 1) // KA
    nbm1 = jnp.maximum(n_b - 1, 0)
    n_p = (n_b + 1) // 2

    def issue(bi, rb, db):
        roff = pl.multiple_of(base + jnp.minimum(bi, nbm1) * KA, KA)
        pltpu.async_copy(rows_hbm.at[pl.ds(roff, KA), :], rb, sem)
        pltpu.async_copy(dlr_hbm.at[pl.ds(roff, KA), :], db, sem)

    def drain2():
        pltpu.make_async_copy(rows_hbm.at[pl.ds(0, KA), :], rb0, sem).wait()
        pltpu.make_async_copy(rows_hbm.at[pl.ds(0, KA), :], db0, sem).wait()

    def apply_batch(bi, rb, db):
        sbase = bi * KA

        def grp(j, _):
            for u in range(16):
                e = 16 * j + u
                valid = sbase + e < count
                dlf = db[e, pl.ds(0, 16)][0]
                dl = jnp.clip(dlf.astype(jnp.int32), 0, DUMMY_ROW)
                dl = jnp.where(valid, dl, DUMMY_ROW)
                for f in range(D // 16):
                    sl = pl.ds(16 * f, 16)
                    acc[dl, sl] = jnp.maximum(acc[dl, sl], rb[e, sl])
            return 0

        lax.fori_loop(0, KA // 16, grp, 0)

    issue(0, rb0, db0)

    def pair(pp, _):
        drain2()
        issue(2 * pp + 1, rb1, db1)
        apply_batch(2 * pp, rb0, db0)
        drain2()
        issue(2 * pp + 2, rb0, db0)
        apply_batch(2 * pp + 1, rb1, db1)
        return 0

    lax.fori_loop(0, n_p, pair, 0)
    drain2()

    pltpu.sync_copy(acc.at[pl.ds(0, RPT)],
                    out_hbm.at[pl.ds(half * N_PAD + rng * RPT, RPT)])


@functools.partial(
    pl.kernel,
    mesh=plsc.VectorSubcoreMesh(core_axis_name="c", subcore_axis_name="s"),
    out_type=jax.ShapeDtypeStruct((N_HALVES * N_PAD, D), jnp.float32),
    scratch_types=[
        pltpu.VMEM((RPT + 8, D), jnp.float32),
        pltpu.VMEM((KA, D), jnp.float32),
        pltpu.VMEM((KA, D), jnp.float32),
        pltpu.VMEM((KA, D), jnp.float32),
        pltpu.VMEM((KA, D), jnp.float32),
        pltpu.VMEM((2, D), jnp.int32),
        pltpu.SemaphoreType.DMA,
    ],
)
def _seg_apply(rows_hbm, dlr_hbm, base_hbm, cnt_hbm, out_hbm,
               acc, rb0, db0, rb1, db1, meta, sem):
    _apply_kernel(rows_hbm, dlr_hbm, base_hbm, cnt_hbm, out_hbm,
                  acc, rb0, db0, rb1, db1, meta, sem)


def kernel(X, edge_index, W1, b1, W2, b2, W3, b3):
    src = edge_index[0]
    dst = edge_index[1]

    srcp = jnp.zeros((E_PAD,), jnp.int32).at[:N_EDGES].set(src)
    dstp = jnp.full((E_PAD,), PAD_DST, jnp.int32).at[:N_EDGES].set(dst)

    w1a = W1[:D] - W1[D:]
    w1b = W1[D:]
    w2a = W2[:D] - W2[D:]
    w2b = W2[D:]
    w3p = jnp.pad(W3, ((0, 0), (0, D - W3.shape[1])))
    b3p = jnp.pad(b3, (0, D - b3.shape[0]))

    xp = jnp.pad(X, ((0, N_PAD - N_NODES), (0, 0)))

    pos2, bases, cnts = _edge_pos(dstp.reshape(E_PAD // SUB, SUB))
    pos = pos2.reshape(-1)

    a1, bb1 = _mm_first(xp, w1a, w1b, b1.reshape(1, D))
    rows1, dlr1 = _edge_scatter(srcp, dstp, pos, bb1)
    s1 = _seg_apply(rows1, dlr1, bases, cnts)
    a2, bb2 = _mm_mid(a1, s1, w2a, w2b, b2.reshape(1, D))
    rows2 = _edge_scatter_rows(srcp, pos, bb2)
    s2 = _seg_apply(rows2, dlr1, bases, cnts)
    out = _mm_post(a2, s2, w3p, b3p.reshape(1, D))
    return out[:N_NODES, :W3.shape[1]]


# speedup vs baseline: 2.6038x; 1.0882x over previous
import functools

import jax
import jax.numpy as jnp
from jax import lax
from jax.experimental import pallas as pl
from jax.experimental.pallas import tpu as pltpu
from jax.experimental.pallas import tpu_sc as plsc

N_NODES = 10000
N_EDGES = 320000
D = 128
N_RANGES = 16
N_HALVES = 2
N_REG = N_RANGES * N_HALVES
RPT = 640
N_PAD = N_RANGES * RPT
DUMMY_ROW = RPT
E_PAD = 323584
HALF_E = E_PAD // 2
CH2 = 2048
N_CH2 = E_PAD // CH2
SUB = 128
N_SUB = CH2 // SUB
EPT = E_PAD // 32
KB = 128
N_EB = EPT // KB
TOTAL_SLOTS = E_PAD + 64 * KB
PAD_DST = 1 << 30
NEG_INF = float("-inf")


_BLK = 1280


def _mm_first_body(x_ref, wa_ref, wb_ref, b_ref, a_ref, bo_ref):
    x = x_ref[...]
    a_ref[...] = (
        jnp.dot(x, wa_ref[...], preferred_element_type=jnp.float32,
                precision=lax.Precision.HIGHEST)
        + b_ref[...]
    )
    bo_ref[...] = jnp.dot(x, wb_ref[...], preferred_element_type=jnp.float32,
                          precision=lax.Precision.HIGHEST)


def _mm_mid_body(a_ref, s0_ref, s1_ref, wa_ref, wb_ref, b_ref,
                 a2_ref, b2_ref):
    s = jnp.maximum(s0_ref[...], s1_ref[...])
    h = jnp.where(s != -jnp.inf, jnp.maximum(a_ref[...] + s, 0.0), 0.0)
    a2_ref[...] = (
        jnp.dot(h, wa_ref[...], preferred_element_type=jnp.float32,
                precision=lax.Precision.HIGHEST)
        + b_ref[...]
    )
    b2_ref[...] = jnp.dot(h, wb_ref[...], preferred_element_type=jnp.float32,
                          precision=lax.Precision.HIGHEST)


def _mm_post_body(a_ref, s0_ref, s1_ref, w_ref, b_ref, o_ref):
    s = jnp.maximum(s0_ref[...], s1_ref[...])
    h = jnp.where(s != -jnp.inf, jnp.maximum(a_ref[...] + s, 0.0), 0.0)
    o_ref[...] = (
        jnp.dot(h, w_ref[...], preferred_element_type=jnp.float32,
                precision=lax.Precision.HIGHEST)
        + b_ref[...]
    )


def _row_spec():
    return pl.BlockSpec((_BLK, D), lambda i: (i, 0))


def _full_spec(shape):
    return pl.BlockSpec(shape, lambda i: (0,) * len(shape))


def _mm_first(x, wa, wb, b):
    return pl.pallas_call(
        _mm_first_body,
        grid=(N_PAD // _BLK,),
        in_specs=[_row_spec(), _full_spec((D, D)), _full_spec((D, D)),
                  _full_spec((1, D))],
        out_specs=[_row_spec(), _row_spec()],
        out_shape=[jax.ShapeDtypeStruct((N_PAD, D), jnp.float32)] * 2,
    )(x, wa, wb, b)


def _mm_mid(a, s2, wa, wb, b):
    return pl.pallas_call(
        _mm_mid_body,
        grid=(N_PAD // _BLK,),
        in_specs=[_row_spec(), _row_spec(), _row_spec(),
                  _full_spec((D, D)), _full_spec((D, D)), _full_spec((1, D))],
        out_specs=[_row_spec(), _row_spec()],
        out_shape=[jax.ShapeDtypeStruct((N_PAD, D), jnp.float32)] * 2,
    )(a, s2[:N_PAD], s2[N_PAD:], wa, wb, b)


def _mm_post(a, s2, w, b):
    return pl.pallas_call(
        _mm_post_body,
        grid=(N_PAD // _BLK,),
        in_specs=[_row_spec(), _row_spec(), _row_spec(),
                  _full_spec((D, D)), _full_spec((1, D))],
        out_specs=_row_spec(),
        out_shape=jax.ShapeDtypeStruct((N_PAD, D), jnp.float32),
    )(a, s2[:N_PAD], s2[N_PAD:], w, b)


def _pos_body(d_ref, pos_ref, base_ref, cnt_ref, carry):
    upper = jnp.where(
        lax.broadcasted_iota(jnp.int32, (SUB, SUB), 0)
        < lax.broadcasted_iota(jnp.int32, (SUB, SUB), 1),
        1.0, 0.0)
    tri64 = jnp.where(
        lax.broadcasted_iota(jnp.int32, (64, 64), 0)
        > lax.broadcasted_iota(jnp.int32, (64, 64), 1),
        1.0, 0.0)
    iota_k = lax.broadcasted_iota(jnp.int32, (64, SUB), 0)
    iota_e = lax.broadcasted_iota(jnp.int32, (SUB,), 0)

    def one_hot(d8, t, s):
        d = d8[s, :]
        epos = (8 * t + s) * SUB + iota_e
        h = (epos >= HALF_E).astype(jnp.int32)
        r = jnp.clip(d // RPT, 0, N_RANGES - 1)
        b = jnp.where(d >= N_PAD, 32, r * N_HALVES + h)
        return (b[None, :] == iota_k).astype(jnp.float32)

    n_grp = (N_CH2 * CH2) // (8 * SUB)
    carry[...] = jnp.zeros((64, 128), jnp.float32)

    def c1(t, _):
        d8 = d_ref[pl.ds(8 * t, 8), :]
        for s in range(8):
            oh = one_hot(d8, t, s)
            tot = jnp.sum(oh, axis=1, keepdims=True)
            carry[...] = carry[...] + jnp.broadcast_to(tot, (64, 128))
        return 0

    lax.fori_loop(0, n_grp, c1, 0)

    counts = carry[...]
    ci = counts.astype(jnp.int32)
    cr = (((ci + KB - 1) // KB) * KB).astype(jnp.float32)
    bases = jnp.dot(tri64, cr)
    base_ref[...] = bases.astype(jnp.int32)
    cnt_ref[...] = ci

    carry[...] = bases

    def c2(t, _):
        d8 = d_ref[pl.ds(8 * t, 8), :]
        outs = []
        for s in range(8):
            oh = one_hot(d8, t, s)
            pre = jnp.dot(oh, upper)
            slot = jnp.sum(oh * (pre + carry[:, :SUB]), axis=0)
            outs.append(slot.astype(jnp.int32)[None, :])
            tot = jnp.sum(oh, axis=1, keepdims=True)
            carry[...] = carry[...] + jnp.broadcast_to(tot, (64, 128))
        pos_ref[pl.ds(8 * t, 8), :] = jnp.concatenate(outs, axis=0)
        return 0

    lax.fori_loop(0, n_grp, c2, 0)


def _edge_pos(d2):
    return pl.pallas_call(
        _pos_body,
        in_specs=[pl.BlockSpec((E_PAD // SUB, SUB), lambda: (0, 0))],
        out_specs=[pl.BlockSpec((E_PAD // SUB, SUB), lambda: (0, 0)),
                   pl.BlockSpec((64, 128), lambda: (0, 0)),
                   pl.BlockSpec((64, 128), lambda: (0, 0))],
        out_shape=[jax.ShapeDtypeStruct((E_PAD // SUB, SUB), jnp.int32),
                   jax.ShapeDtypeStruct((64, 128), jnp.int32),
                   jax.ShapeDtypeStruct((64, 128), jnp.int32)],
        scratch_shapes=[pltpu.VMEM((64, 128), jnp.float32)],
    )(d2)


def _scatter_kernel(src_hbm, dst_hbm, pos_hbm, b_hbm, rows_hbm, dlr_hbm,
                    srcv0, dstv0, posv0, brow0, dlbuf0,
                    srcv1, dstv1, posv1, brow1, dlbuf1,
                    sg0, sg1, ss0, ss1):
    wid = lax.axis_index("c") * 16 + lax.axis_index("s")
    tbase = wid * EPT
    ones16 = jnp.full((16,), 1.0, dtype=jnp.float32)

    def meta(bi, sv, dv, pv):
        off = tbase + jnp.minimum(bi, N_EB - 1) * KB
        pltpu.sync_copy(src_hbm.at[pl.ds(off, KB)], sv)
        pltpu.sync_copy(dst_hbm.at[pl.ds(off, KB)], dv)
        pltpu.sync_copy(pos_hbm.at[pl.ds(off, KB)], pv)

    def build(dv, dlbuf):
        def grp(j, _):
            d16 = dv[pl.ds(16 * j, 16)]
            dlv = (d16 % RPT).astype(jnp.float32)
            for u in range(16):
                splat = ones16 * dlv[u]
                for f in range(D // 16):
                    dlbuf[16 * j + u, pl.ds(16 * f, 16)] = splat
            return 0

        lax.fori_loop(0, KB // 16, grp, 0)

    def gwait(sem, brow):
        pltpu.make_async_copy(b_hbm.at[pl.ds(0, KB), :], brow, sem).wait()

    def swait(sem, brow, dlbuf):
        pltpu.make_async_copy(b_hbm.at[pl.ds(0, KB), :], brow, sem).wait()
        pltpu.make_async_copy(b_hbm.at[pl.ds(0, KB), :], dlbuf, sem).wait()

    meta(0, srcv0, dstv0, posv0)
    pltpu.async_copy(b_hbm.at[srcv0], brow0, sg0)

    def pair(pp, _):
        meta(2 * pp + 1, srcv1, dstv1, posv1)
        pltpu.async_copy(b_hbm.at[srcv1], brow1, sg1)
        build(dstv0, dlbuf0)
        gwait(sg0, brow0)
        pltpu.async_copy(brow0, rows_hbm.at[posv0], ss0)
        pltpu.async_copy(dlbuf0, dlr_hbm.at[posv0], ss0)
        swait(ss0, brow0, dlbuf0)
        meta(2 * pp + 2, srcv0, dstv0, posv0)
        pltpu.async_copy(b_hbm.at[srcv0], brow0, sg0)
        build(dstv1, dlbuf1)
        gwait(sg1, brow1)
        pltpu.async_copy(brow1, rows_hbm.at[posv1], ss1)
        pltpu.async_copy(dlbuf1, dlr_hbm.at[posv1], ss1)
        swait(ss1, brow1, dlbuf1)
        return 0

    lax.fori_loop(0, (N_EB + 1) // 2, pair, 0)
    gwait(sg0, brow0)


@functools.partial(
    pl.kernel,
    mesh=plsc.VectorSubcoreMesh(core_axis_name="c", subcore_axis_name="s"),
    out_type=[jax.ShapeDtypeStruct((TOTAL_SLOTS, D), jnp.float32),
              jax.ShapeDtypeStruct((TOTAL_SLOTS, D), jnp.float32)],
    scratch_types=[
        pltpu.VMEM((KB,), jnp.int32),
        pltpu.VMEM((KB,), jnp.int32),
        pltpu.VMEM((KB,), jnp.int32),
        pltpu.VMEM((KB, D), jnp.float32),
        pltpu.VMEM((KB, D), jnp.float32),
        pltpu.VMEM((KB,), jnp.int32),
        pltpu.VMEM((KB,), jnp.int32),
        pltpu.VMEM((KB,), jnp.int32),
        pltpu.VMEM((KB, D), jnp.float32),
        pltpu.VMEM((KB, D), jnp.float32),
        pltpu.SemaphoreType.DMA,
        pltpu.SemaphoreType.DMA,
        pltpu.SemaphoreType.DMA,
        pltpu.SemaphoreType.DMA,
    ],
)
def _edge_scatter(src_hbm, dst_hbm, pos_hbm, b_hbm, rows_hbm, dlr_hbm,
                  *scratch):
    _scatter_kernel(src_hbm, dst_hbm, pos_hbm, b_hbm, rows_hbm, dlr_hbm,
                    *scratch)


def _scatter_rows_kernel(src_hbm, pos_hbm, b_hbm, rows_hbm,
                         srcv0, posv0, brow0, srcv1, posv1, brow1,
                         sg0, sg1, ss0, ss1):
    wid = lax.axis_index("c") * 16 + lax.axis_index("s")
    tbase = wid * EPT

    def meta(bi, sv, pv):
        off = tbase + jnp.minimum(bi, N_EB - 1) * KB
        pltpu.sync_copy(src_hbm.at[pl.ds(off, KB)], sv)
        pltpu.sync_copy(pos_hbm.at[pl.ds(off, KB)], pv)

    def dwait(sem, buf):
        pltpu.make_async_copy(b_hbm.at[pl.ds(0, KB), :], buf, sem).wait()

    meta(0, srcv0, posv0)
    pltpu.async_copy(b_hbm.at[srcv0], brow0, sg0)

    def pair(pp, _):
        meta(2 * pp + 1, srcv1, posv1)
        pltpu.async_copy(b_hbm.at[srcv1], brow1, sg1)
        dwait(sg0, brow0)
        pltpu.async_copy(brow0, rows_hbm.at[posv0], ss0)
        dwait(ss0, brow0)
        meta(2 * pp + 2, srcv0, posv0)
        pltpu.async_copy(b_hbm.at[srcv0], brow0, sg0)
        dwait(sg1, brow1)
        pltpu.async_copy(brow1, rows_hbm.at[posv1], ss1)
        dwait(ss1, brow1)
        return 0

    lax.fori_loop(0, (N_EB + 1) // 2, pair, 0)
    dwait(sg0, brow0)


@functools.partial(
    pl.kernel,
    mesh=plsc.VectorSubcoreMesh(core_axis_name="c", subcore_axis_name="s"),
    out_type=jax.ShapeDtypeStruct((TOTAL_SLOTS, D), jnp.float32),
    scratch_types=[
        pltpu.VMEM((KB,), jnp.int32),
        pltpu.VMEM((KB,), jnp.int32),
        pltpu.VMEM((KB, D), jnp.float32),
        pltpu.VMEM((KB,), jnp.int32),
        pltpu.VMEM((KB,), jnp.int32),
        pltpu.VMEM((KB, D), jnp.float32),
        pltpu.SemaphoreType.DMA,
        pltpu.SemaphoreType.DMA,
        pltpu.SemaphoreType.DMA,
        pltpu.SemaphoreType.DMA,
    ],
)
def _edge_scatter_rows(src_hbm, pos_hbm, b_hbm, rows_hbm, *scratch):
    _scatter_rows_kernel(src_hbm, pos_hbm, b_hbm, rows_hbm, *scratch)


KA = 64


def _apply_kernel(rows_hbm, dlr_hbm, base_hbm, cnt_hbm, out_hbm,
                  acc, rb0, db0, rb1, db1, meta, sem):
    wid = lax.axis_index("c") * 16 + lax.axis_index("s")
    rng = wid % N_RANGES
    half = wid // N_RANGES
    region = rng * N_HALVES + half

    pltpu.sync_copy(base_hbm.at[pl.ds(region, 1), :], meta.at[pl.ds(0, 1), :])
    pltpu.sync_copy(cnt_hbm.at[pl.ds(region, 1), :], meta.at[pl.ds(1, 1), :])
    base = pl.multiple_of(meta[0, pl.ds(0, 16)][0], KB)
    count = meta[1, pl.ds(0, 16)][0]

    neg = jnp.full((16,), NEG_INF, dtype=jnp.float32)

    def init_body(r, _):
        for f in range(D // 16):
            acc[r, pl.ds(16 * f, 16)] = neg
        return 0

    lax.fori_loop(0, RPT + 8, init_body, 0)

    n_b = (count + KA - 1) // KA
    nbm1 = jnp.maximum(n_b - 1, 0)
    n_p = (n_b + 1) // 2

    def issue(bi, rb, db):
        roff = pl.multiple_of(base + jnp.minimum(bi, nbm1) * KA, KA)
        pltpu.async_copy(rows_hbm.at[pl.ds(roff, KA), :], rb, sem)
        pltpu.async_copy(dlr_hbm.at[pl.ds(roff, KA), :], db, sem)

    def drain2():
        pltpu.make_async_copy(rows_hbm.at[pl.ds(0, KA), :], rb0, sem).wait()
        pltpu.make_async_copy(rows_hbm.at[pl.ds(0, KA), :], db0, sem).wait()

    def apply_batch(bi, rb, db):
        sbase = bi * KA

        def grp(j, _):
            for u in range(16):
                e = 16 * j + u
                valid = sbase + e < count
                dlf = db[e, pl.ds(0, 16)][0]
                dl = jnp.clip(dlf.astype(jnp.int32), 0, DUMMY_ROW)
                dl = jnp.where(valid, dl, DUMMY_ROW)
                for f in range(D // 16):
                    sl = pl.ds(16 * f, 16)
                    acc[dl, sl] = jnp.maximum(acc[dl, sl], rb[e, sl])
            return 0

        lax.fori_loop(0, KA // 16, grp, 0)

    issue(0, rb0, db0)

    def pair(pp, _):
        drain2()
        issue(2 * pp + 1, rb1, db1)
        apply_batch(2 * pp, rb0, db0)
        drain2()
        issue(2 * pp + 2, rb0, db0)
        apply_batch(2 * pp + 1, rb1, db1)
        return 0

    lax.fori_loop(0, n_p, pair, 0)
    drain2()

    pltpu.sync_copy(acc.at[pl.ds(0, RPT)],
                    out_hbm.at[pl.ds(half * N_PAD + rng * RPT, RPT)])


@functools.partial(
    pl.kernel,
    mesh=plsc.VectorSubcoreMesh(core_axis_name="c", subcore_axis_name="s"),
    out_type=jax.ShapeDtypeStruct((N_HALVES * N_PAD, D), jnp.float32),
    scratch_types=[
        pltpu.VMEM((RPT + 8, D), jnp.float32),
        pltpu.VMEM((KA, D), jnp.float32),
        pltpu.VMEM((KA, D), jnp.float32),
        pltpu.VMEM((KA, D), jnp.float32),
        pltpu.VMEM((KA, D), jnp.float32),
        pltpu.VMEM((2, D), jnp.int32),
        pltpu.SemaphoreType.DMA,
    ],
)
def _seg_apply(rows_hbm, dlr_hbm, base_hbm, cnt_hbm, out_hbm,
               acc, rb0, db0, rb1, db1, meta, sem):
    _apply_kernel(rows_hbm, dlr_hbm, base_hbm, cnt_hbm, out_hbm,
                  acc, rb0, db0, rb1, db1, meta, sem)


def kernel(X, edge_index, W1, b1, W2, b2, W3, b3):
    src = edge_index[0]
    dst = edge_index[1]

    srcp = jnp.zeros((E_PAD,), jnp.int32).at[:N_EDGES].set(src)
    dstp = jnp.full((E_PAD,), PAD_DST, jnp.int32).at[:N_EDGES].set(dst)

    w1a = W1[:D] - W1[D:]
    w1b = W1[D:]
    w2a = W2[:D] - W2[D:]
    w2b = W2[D:]
    w3p = jnp.pad(W3, ((0, 0), (0, D - W3.shape[1])))
    b3p = jnp.pad(b3, (0, D - b3.shape[0]))

    xp = jnp.pad(X, ((0, N_PAD - N_NODES), (0, 0)))

    pos2, bases, cnts = _edge_pos(dstp.reshape(E_PAD // SUB, SUB))
    pos = pos2.reshape(-1)

    a1, bb1 = _mm_first(xp, w1a, w1b, b1.reshape(1, D))
    rows1, dlr1 = _edge_scatter(srcp, dstp, pos, bb1)
    s1 = _seg_apply(rows1, dlr1, bases, cnts)
    a2, bb2 = _mm_mid(a1, s1, w2a, w2b, b2.reshape(1, D))
    rows2 = _edge_scatter_rows(srcp, pos, bb2)
    s2 = _seg_apply(rows2, dlr1, bases, cnts)
    out = _mm_post(a2, s2, w3p, b3p.reshape(1, D))
    return out[:N_NODES, :W3.shape[1]]
